# baseline (speedup 1.0000x reference)
import functools
import math

import jax
import jax.numpy as jnp
from jax import lax
from jax.experimental import pallas as pl
from jax.experimental.pallas import tpu as pltpu

_EPS = 1e-5
_INV_SQRT2 = 1.0 / math.sqrt(2.0)

_SEL = (((0,), (1, 2)), ((0, 1), (2,)))


def _build_taps(x_ref, H, W):
    L = H * W
    x = jnp.pad(x_ref[0].astype(jnp.bfloat16), ((0, 0), (W + 1, W + 1)))
    jmod = lax.broadcasted_iota(jnp.int32, (1, L), 1) % W
    m_first = jmod == 0
    m_last = jmod == (W - 1)
    taps = {}
    for a in range(3):
        for b in range(3):
            t = x[:, W * a + b:W * a + b + L]
            if b == 0:
                t = jnp.where(m_first, jnp.zeros_like(t), t)
            elif b == 2:
                t = jnp.where(m_last, jnp.zeros_like(t), t)
            taps[(a, b)] = t
    return taps


def _conv_parity(taps, w_ref, py, px):
    conv = None
    for dr in range(2):
        for dc in range(2):
            wm = w_ref[(py * 2 + px) * 4 + dr * 2 + dc]
            d = lax.dot_general(wm, taps[(py + dr, px + dc)],
                                (((1,), (0,)), ((), ())),
                                preferred_element_type=jnp.float32)
            conv = d if conv is None else conv + d
    return conv


def _stats_kernel(x_ref, w_ref, sum_ref, ssq_ref, *, H, W):
    @pl.when(pl.program_id(0) == 0)
    def _():
        sum_ref[...] = jnp.zeros_like(sum_ref)
        ssq_ref[...] = jnp.zeros_like(ssq_ref)

    taps = _build_taps(x_ref, H, W)
    s_acc = None
    q_acc = None
    for py in range(2):
        for px in range(2):
            conv = _conv_parity(taps, w_ref, py, px)
            s = jnp.sum(conv, axis=1, keepdims=True)
            q = jnp.sum(conv * conv, axis=1, keepdims=True)
            s_acc = s if s_acc is None else s_acc + s
            q_acc = q if q_acc is None else q_acc + q
    sum_ref[...] += s_acc
    ssq_ref[...] += q_acc


def _apply_kernel(x_ref, w_ref, scale_ref, shift_ref, o_ref, *, H, W):
    taps = _build_taps(x_ref, H, W)
    scale = scale_ref[...]
    shift = shift_ref[...]
    for py in range(2):
        for px in range(2):
            y = _conv_parity(taps, w_ref, py, px) * scale + shift
            o_ref[0, py * 2 + px] = 0.5 * y * (1.0 + lax.erf(y * _INV_SQRT2))


def _fold_weights(weight):
    w = weight.astype(jnp.float32)
    mats = []
    for py in range(2):
        for px in range(2):
            for dr in range(2):
                for dc in range(2):
                    m = None
                    for kh in _SEL[py][dr]:
                        for kw in _SEL[px][dc]:
                            t = w[:, :, kh, kw]
                            m = t if m is None else m + t
                    mats.append(m)
    return jnp.stack(mats).astype(jnp.bfloat16)


def kernel(x, weight, bias, gamma, beta):
    del bias
    N, Cin, H, W = x.shape
    Cout = weight.shape[0]
    L = H * W

    w16 = _fold_weights(weight)
    xf = x.reshape(N, Cin, L)

    sum_p, ssq_p = pl.pallas_call(
        functools.partial(_stats_kernel, H=H, W=W),
        grid=(N,),
        in_specs=[pl.BlockSpec((1, Cin, L), lambda n: (n, 0, 0)),
                  pl.BlockSpec((16, Cout, Cin), lambda n: (0, 0, 0))],
        out_specs=(pl.BlockSpec((Cout, 1), lambda n: (0, 0)),
                   pl.BlockSpec((Cout, 1), lambda n: (0, 0))),
        out_shape=(jax.ShapeDtypeStruct((Cout, 1), jnp.float32),
                   jax.ShapeDtypeStruct((Cout, 1), jnp.float32)),
        compiler_params=pltpu.CompilerParams(
            dimension_semantics=("arbitrary",)),
    )(xf, w16)

    count = jnp.float32(N * 4 * L)
    mean = sum_p[:, 0] / count
    var = ssq_p[:, 0] / count - mean * mean
    scale = gamma.astype(jnp.float32) * lax.rsqrt(var + _EPS)
    shift = beta.astype(jnp.float32) - mean * scale

    out_par = pl.pallas_call(
        functools.partial(_apply_kernel, H=H, W=W),
        grid=(N,),
        in_specs=[pl.BlockSpec((1, Cin, L), lambda n: (n, 0, 0)),
                  pl.BlockSpec((16, Cout, Cin), lambda n: (0, 0, 0)),
                  pl.BlockSpec((Cout, 1), lambda n: (0, 0)),
                  pl.BlockSpec((Cout, 1), lambda n: (0, 0))],
        out_specs=pl.BlockSpec((1, 4, Cout, L), lambda n: (n, 0, 0, 0)),
        out_shape=jax.ShapeDtypeStruct((N, 4, Cout, L), jnp.float32),
        compiler_params=pltpu.CompilerParams(
            dimension_semantics=("arbitrary",)),
    )(xf, w16, scale.reshape(Cout, 1), shift.reshape(Cout, 1))

    p = out_par.reshape(N, 2, 2, Cout, H, W)
    out = None
    for py in range(2):
        for px in range(2):
            d = lax.pad(p[:, py, px], jnp.float32(0),
                        ((0, 0, 0), (0, 0, 0), (py, 1 - py, 1), (px, 1 - px, 1)))
            out = d if out is None else out + d
    return out


# speedup vs baseline: 8.1265x; 8.1265x over previous
import functools
import math

import jax
import jax.numpy as jnp
from jax import lax
from jax.experimental import pallas as pl
from jax.experimental.pallas import tpu as pltpu

_EPS = 1e-5
_INV_SQRT2 = 1.0 / math.sqrt(2.0)

_SEL = (((0,), (1, 2)), ((0, 1), (2,)))


def _build_taps(x_ref, H, W):
    L = H * W
    x = jnp.pad(x_ref[0].astype(jnp.bfloat16), ((0, 0), (W + 1, W + 1)))
    jmod = lax.broadcasted_iota(jnp.int32, (1, L), 1) % W
    m_first = jmod == 0
    m_last = jmod == (W - 1)
    taps = {}
    for a in range(3):
        for b in range(3):
            t = x[:, W * a + b:W * a + b + L]
            if b == 0:
                t = jnp.where(m_first, jnp.zeros_like(t), t)
            elif b == 2:
                t = jnp.where(m_last, jnp.zeros_like(t), t)
            taps[(a, b)] = t
    return taps


def _conv_parity(taps, w_ref, py, px):
    conv = None
    for dr in range(2):
        for dc in range(2):
            wm = w_ref[(py * 2 + px) * 4 + dr * 2 + dc]
            d = lax.dot_general(wm, taps[(py + dr, px + dc)],
                                (((1,), (0,)), ((), ())),
                                preferred_element_type=jnp.float32)
            conv = d if conv is None else conv + d
    return conv


def _stats_kernel(x_ref, w_ref, sum_ref, ssq_ref, *, H, W):
    @pl.when(pl.program_id(0) == 0)
    def _():
        sum_ref[...] = jnp.zeros_like(sum_ref)
        ssq_ref[...] = jnp.zeros_like(ssq_ref)

    taps = _build_taps(x_ref, H, W)
    s_acc = None
    q_acc = None
    for py in range(2):
        for px in range(2):
            conv = _conv_parity(taps, w_ref, py, px)
            s = jnp.sum(conv, axis=1, keepdims=True)
            q = jnp.sum(conv * conv, axis=1, keepdims=True)
            s_acc = s if s_acc is None else s_acc + s
            q_acc = q if q_acc is None else q_acc + q
    sum_ref[...] += s_acc
    ssq_ref[...] += q_acc


def _apply_kernel(x_ref, w_ref, scale_ref, shift_ref, o_ref, *, H, W):
    taps = _build_taps(x_ref, H, W)
    scale = scale_ref[...]
    shift = shift_ref[...]
    for py in range(2):
        for px in range(2):
            y = _conv_parity(taps, w_ref, py, px) * scale + shift
            g = 0.5 * y * (1.0 + lax.erf(y * _INV_SQRT2))
            o_ref[0, py * 2 + px] = g.astype(jnp.bfloat16)


def _fold_weights(weight):
    w = weight.astype(jnp.float32)
    mats = []
    for py in range(2):
        for px in range(2):
            for dr in range(2):
                for dc in range(2):
                    m = None
                    for kh in _SEL[py][dr]:
                        for kw in _SEL[px][dc]:
                            t = w[:, :, kh, kw]
                            m = t if m is None else m + t
                    mats.append(m)
    return jnp.stack(mats).astype(jnp.bfloat16)


def kernel(x, weight, bias, gamma, beta):
    del bias
    N, Cin, H, W = x.shape
    Cout = weight.shape[0]
    L = H * W

    w16 = _fold_weights(weight)
    xf = x.reshape(N, Cin, L)

    sum_p, ssq_p = pl.pallas_call(
        functools.partial(_stats_kernel, H=H, W=W),
        grid=(N,),
        in_specs=[pl.BlockSpec((1, Cin, L), lambda n: (n, 0, 0)),
                  pl.BlockSpec((16, Cout, Cin), lambda n: (0, 0, 0))],
        out_specs=(pl.BlockSpec((Cout, 1), lambda n: (0, 0)),
                   pl.BlockSpec((Cout, 1), lambda n: (0, 0))),
        out_shape=(jax.ShapeDtypeStruct((Cout, 1), jnp.float32),
                   jax.ShapeDtypeStruct((Cout, 1), jnp.float32)),
        compiler_params=pltpu.CompilerParams(
            dimension_semantics=("arbitrary",)),
    )(xf, w16)

    count = jnp.float32(N * 4 * L)
    mean = sum_p[:, 0] / count
    var = ssq_p[:, 0] / count - mean * mean
    scale = gamma.astype(jnp.float32) * lax.rsqrt(var + _EPS)
    shift = beta.astype(jnp.float32) - mean * scale

    out_par = pl.pallas_call(
        functools.partial(_apply_kernel, H=H, W=W),
        grid=(N,),
        in_specs=[pl.BlockSpec((1, Cin, L), lambda n: (n, 0, 0)),
                  pl.BlockSpec((16, Cout, Cin), lambda n: (0, 0, 0)),
                  pl.BlockSpec((Cout, 1), lambda n: (0, 0)),
                  pl.BlockSpec((Cout, 1), lambda n: (0, 0))],
        out_specs=pl.BlockSpec((1, 4, Cout, L), lambda n: (n, 0, 0, 0)),
        out_shape=jax.ShapeDtypeStruct((N, 4, Cout, L), jnp.bfloat16),
        compiler_params=pltpu.CompilerParams(
            dimension_semantics=("arbitrary",)),
    )(xf, w16, scale.reshape(Cout, 1), shift.reshape(Cout, 1))

    out = out_par.reshape(N, 2, 2, Cout, H, W)
    out = out.transpose(0, 3, 4, 1, 5, 2).reshape(N, Cout, 2 * H, 2 * W)
    return out.astype(jnp.float32)
